# baseline (speedup 1.0000x reference)
import functools

import jax
import jax.numpy as jnp
from jax import lax
from jax.experimental import pallas as pl
from jax.experimental.pallas import tpu as pltpu
from jax.experimental.pallas import tpu_sc as plsc

ITERATIONS = 3
AD = 5
FOV = 11
MAX_DIST = FOV // 2
NPARAM = FOV * FOV
A = 128
L = 16
NCHUNK = A // L
AP = A + L
CTLEN = NPARAM * AD * AD
INVALID = CTLEN
CTPAD = 3056
MAXC = A * NCHUNK
NW = 32
BPW = 256 // NW


def _sc_body(ct_hbm, lgt_hbm, ill_hbm, q0_hbm, py_hbm, px_hbm, out_hbm,
             ct_v, lgt_v, ill_v, q0_v, qlog_v, qp_v, corr_v, py_v, px_v,
             pb_v, hd_v, ctail_v, cslot_v):
    wid = lax.axis_index("s") * 2 + lax.axis_index("c")
    pltpu.sync_copy(ct_hbm, ct_v)
    lanes = lax.iota(jnp.int32, L)

    def batch_body(bi, carry):
        b = wid * BPW + bi
        pltpu.sync_copy(lgt_hbm.at[b], lgt_v)
        pltpu.sync_copy(ill_hbm.at[b], ill_v)
        pltpu.sync_copy(q0_hbm.at[b], q0_v)
        pltpu.sync_copy(py_hbm.at[b], py_v)
        pltpu.sync_copy(px_hbm.at[b], px_v)

        for a in range(AD):
            for c in range(NCHUNK):
                s = pl.ds(c * L, L)
                qlog_v[a, s] = q0_v[a, s] - ill_v[a, s] * 1e10

        for c in range(MAXC // L):
            s = pl.ds(c * L, L)
            ctail_v[s] = jnp.zeros((L,), jnp.int32)
            cslot_v[s] = jnp.full((L,), NCHUNK, jnp.int32)

        def build_body(j, mv):
            jf = jnp.full((L,), j, jnp.int32)
            yj = plsc.load_gather(py_v, [jf])
            xj = plsc.load_gather(px_v, [jf])
            for ic in range(NCHUNK + 1):
                hd_v[j, pl.ds(ic * L, L)] = A + lanes
            cntv = jnp.zeros((L,), jnp.int32)
            for ic in range(NCHUNK):
                s = pl.ds(ic * L, L)
                dy = jnp.abs(py_v[s] - yj)
                dx = jnp.abs(px_v[s] - xj)
                nb = jnp.maximum(dy, dx) <= MAX_DIST
                pb = ((dy + MAX_DIST) * FOV + (dx + MAX_DIST)) * (AD * AD)
                pb_v[j, s] = jnp.where(nb, pb, INVALID)
                pos = cntv + jnp.cumsum(nb.astype(jnp.int32)) - 1
                plsc.store_scatter(hd_v, [jf, pos], ic * L + lanes, mask=nb)
                cntv = cntv + plsc.all_reduce_population_count(nb)
            pb_v[j, pl.ds(A, L)] = jnp.full((L,), INVALID, jnp.int32)
            nchv = (cntv + (L - 1)) // L
            plsc.store_scatter(ctail_v, [mv + lanes], jf, mask=lanes < nchv)
            plsc.store_scatter(cslot_v, [mv + lanes], lanes, mask=lanes < nchv)
            return mv + nchv

        mv = lax.fori_loop(0, A, build_body, jnp.zeros((L,), jnp.int32))
        nhalf = (mv[0] + 3) // 4

        for _ in range(ITERATIONS):
            for c in range(NCHUNK):
                s = pl.ds(c * L, L)
                qs = [qlog_v[a, s] for a in range(AD)]
                m = qs[0]
                for a in range(1, AD):
                    m = jnp.maximum(m, qs[a])
                es = [jnp.exp(q - m) for q in qs]
                tot = es[0]
                for a in range(1, AD):
                    tot = tot + es[a]
                for a in range(AD):
                    qp_v[a, s] = es[a] / tot

            for a in range(AD):
                for c in range(NCHUNK + 1):
                    corr_v[a, pl.ds(c * L, L)] = jnp.zeros((L,), jnp.float32)

            def chunk_body(ci, cc):
                for e in range(4):
                    ef = jnp.full((L,), 4 * ci + e, jnp.int32)
                    jf = plsc.load_gather(ctail_v, [ef])
                    slot = plsc.load_gather(cslot_v, [ef])
                    hv = plsc.load_gather(hd_v, [jf, slot * L + lanes])
                    basev = plsc.load_gather(pb_v, [jf, hv])
                    qsp = [plsc.load_gather(
                        qp_v, [jnp.full((L,), a2, jnp.int32), jf])
                        for a2 in range(AD)]
                    accs = [jnp.zeros((L,), jnp.float32) for _ in range(AD)]
                    for a2 in range(AD):
                        for a in range(AD):
                            cv = plsc.load_gather(
                                ct_v, [basev + (a * AD + a2)])
                            accs[a] = accs[a] + cv * qsp[a2]
                    for a in range(AD):
                        plsc.addupdate_scatter(
                            corr_v, [jnp.full((L,), a, jnp.int32), hv],
                            accs[a])
                return cc

            lax.fori_loop(0, nhalf, chunk_body, 0)

            for a in range(AD):
                for c in range(NCHUNK):
                    s = pl.ds(c * L, L)
                    qlog_v[a, s] = (lgt_v[a, s] + corr_v[a, s]
                                    - ill_v[a, s] * 1e10)

        pltpu.sync_copy(qlog_v, out_hbm.at[b])
        return carry

    lax.fori_loop(0, BPW, batch_body, 0)


@jax.jit
def kernel(logits, illegal_action_masks, curr_positions, correlation_params):
    B, A_, AD_ = logits.shape
    lgt = jnp.swapaxes(logits, 1, 2)
    illt = jnp.swapaxes(illegal_action_masks, 1, 2)
    q0 = jax.random.uniform(jax.random.key(1), logits.shape,
                            dtype=logits.dtype)
    q0t = jnp.swapaxes(q0, 1, 2)
    pos = curr_positions.astype(jnp.int32)
    py = pos[:, :, 0]
    px = pos[:, :, 1]
    ct = jnp.pad(correlation_params.reshape(-1), (0, CTPAD - CTLEN))

    mesh = plsc.VectorSubcoreMesh(core_axis_name="c", subcore_axis_name="s")
    run = functools.partial(
        pl.kernel,
        mesh=mesh,
        compiler_params=pltpu.CompilerParams(needs_layout_passes=False),
        out_type=jax.ShapeDtypeStruct((B, AD_, A_), jnp.float32),
        scratch_types=[
            pltpu.VMEM((CTPAD,), jnp.float32),
            pltpu.VMEM((AD, A), jnp.float32),
            pltpu.VMEM((AD, A), jnp.float32),
            pltpu.VMEM((AD, A), jnp.float32),
            pltpu.VMEM((AD, A), jnp.float32),
            pltpu.VMEM((AD, A), jnp.float32),
            pltpu.VMEM((AD, AP), jnp.float32),
            pltpu.VMEM((A,), jnp.int32),
            pltpu.VMEM((A,), jnp.int32),
            pltpu.VMEM((A, AP), jnp.int32),
            pltpu.VMEM((A, AP), jnp.int32),
            pltpu.VMEM((MAXC,), jnp.int32),
            pltpu.VMEM((MAXC,), jnp.int32),
        ],
    )(_sc_body)
    outt = run(ct, lgt, illt, q0t, py, px)
    return jnp.swapaxes(outt, 1, 2)


# speedup vs baseline: 1.0041x; 1.0041x over previous
import functools

import jax
import jax.numpy as jnp
from jax import lax
from jax.experimental import pallas as pl
from jax.experimental.pallas import tpu as pltpu
from jax.experimental.pallas import tpu_sc as plsc

ITERATIONS = 3
AD = 5
FOV = 11
MAX_DIST = FOV // 2
NPARAM = FOV * FOV
A = 128
L = 16
NCHUNK = A // L
AP = A + L
CTLEN = NPARAM * AD * AD
INVALID = CTLEN
CTPAD = 3056
MAXC = A * NCHUNK
NW = 32
BPW = 256 // NW


def _sc_body(ct_hbm, lgt_hbm, ill_hbm, q0_hbm, py_hbm, px_hbm, out_hbm,
             ct_v, lgt_v, ill_v, q0_v, qlog_v, qp_v, corr_v, py_v, px_v,
             pb_v, hd_v, ctail_v, cslot_v):
    wid = lax.axis_index("s") * 2 + lax.axis_index("c")
    pltpu.sync_copy(ct_hbm, ct_v)
    lanes = lax.iota(jnp.int32, L)

    def batch_body(bi, carry):
        b = wid * BPW + bi
        pltpu.sync_copy(lgt_hbm.at[b], lgt_v)
        pltpu.sync_copy(ill_hbm.at[b], ill_v)
        pltpu.sync_copy(q0_hbm.at[b], q0_v)
        pltpu.sync_copy(py_hbm.at[b], py_v)
        pltpu.sync_copy(px_hbm.at[b], px_v)

        for a in range(AD):
            for c in range(NCHUNK):
                s = pl.ds(c * L, L)
                qlog_v[a, s] = q0_v[a, s] - ill_v[a, s] * 1e10

        for c in range(MAXC // L):
            s = pl.ds(c * L, L)
            ctail_v[s] = jnp.zeros((L,), jnp.int32)
            cslot_v[s] = jnp.full((L,), NCHUNK, jnp.int32)

        def build_body(j, mv):
            jf = jnp.full((L,), j, jnp.int32)
            yj = plsc.load_gather(py_v, [jf])
            xj = plsc.load_gather(px_v, [jf])
            for ic in range(NCHUNK + 1):
                hd_v[j, pl.ds(ic * L, L)] = A + lanes
            cntv = jnp.zeros((L,), jnp.int32)
            for ic in range(NCHUNK):
                s = pl.ds(ic * L, L)
                dy = jnp.abs(py_v[s] - yj)
                dx = jnp.abs(px_v[s] - xj)
                nb = jnp.maximum(dy, dx) <= MAX_DIST
                pb = ((dy + MAX_DIST) * FOV + (dx + MAX_DIST)) * (AD * AD)
                pb_v[j, s] = jnp.where(nb, pb, INVALID)
                pos = cntv + jnp.cumsum(nb.astype(jnp.int32)) - 1
                plsc.store_scatter(hd_v, [jf, pos], ic * L + lanes, mask=nb)
                cntv = cntv + plsc.all_reduce_population_count(nb)
            pb_v[j, pl.ds(A, L)] = jnp.full((L,), INVALID, jnp.int32)
            nchv = (cntv + (L - 1)) // L
            plsc.store_scatter(ctail_v, [mv + lanes], jf, mask=lanes < nchv)
            plsc.store_scatter(cslot_v, [mv + lanes], lanes, mask=lanes < nchv)
            return mv + nchv

        mv = lax.fori_loop(0, A, build_body, jnp.zeros((L,), jnp.int32))
        nhalf = (mv[0] + 1) // 2

        for _ in range(ITERATIONS):
            for c in range(NCHUNK):
                s = pl.ds(c * L, L)
                qs = [qlog_v[a, s] for a in range(AD)]
                m = qs[0]
                for a in range(1, AD):
                    m = jnp.maximum(m, qs[a])
                es = [jnp.exp(q - m) for q in qs]
                tot = es[0]
                for a in range(1, AD):
                    tot = tot + es[a]
                for a in range(AD):
                    qp_v[a, s] = es[a] / tot

            for a in range(AD):
                for c in range(NCHUNK + 1):
                    corr_v[a, pl.ds(c * L, L)] = jnp.zeros((L,), jnp.float32)

            def chunk_body(ci, cc):
                for e in range(2):
                    ef = jnp.full((L,), 2 * ci + e, jnp.int32)
                    jf = plsc.load_gather(ctail_v, [ef])
                    slot = plsc.load_gather(cslot_v, [ef])
                    hv = plsc.load_gather(hd_v, [jf, slot * L + lanes])
                    basev = plsc.load_gather(pb_v, [jf, hv])
                    qsp = [plsc.load_gather(
                        qp_v, [jnp.full((L,), a2, jnp.int32), jf])
                        for a2 in range(AD)]
                    accs = [jnp.zeros((L,), jnp.float32) for _ in range(AD)]
                    for a2 in range(AD):
                        for a in range(AD):
                            cv = plsc.load_gather(
                                ct_v, [basev + (a * AD + a2)])
                            accs[a] = accs[a] + cv * qsp[a2]
                    for a in range(AD):
                        plsc.addupdate_scatter(
                            corr_v, [jnp.full((L,), a, jnp.int32), hv],
                            accs[a])
                return cc

            lax.fori_loop(0, nhalf, chunk_body, 0)

            for a in range(AD):
                for c in range(NCHUNK):
                    s = pl.ds(c * L, L)
                    qlog_v[a, s] = (lgt_v[a, s] + corr_v[a, s]
                                    - ill_v[a, s] * 1e10)

        pltpu.sync_copy(qlog_v, out_hbm.at[b])
        return carry

    lax.fori_loop(0, BPW, batch_body, 0)


@jax.jit
def kernel(logits, illegal_action_masks, curr_positions, correlation_params):
    B, A_, AD_ = logits.shape
    lgt = jnp.swapaxes(logits, 1, 2)
    illt = jnp.swapaxes(illegal_action_masks, 1, 2)
    q0 = jax.random.uniform(jax.random.key(1), logits.shape,
                            dtype=logits.dtype)
    q0t = jnp.swapaxes(q0, 1, 2)
    pos = curr_positions.astype(jnp.int32)
    py = pos[:, :, 0]
    px = pos[:, :, 1]
    ct = jnp.pad(correlation_params.reshape(-1), (0, CTPAD - CTLEN))

    mesh = plsc.VectorSubcoreMesh(core_axis_name="c", subcore_axis_name="s")
    run = functools.partial(
        pl.kernel,
        mesh=mesh,
        compiler_params=pltpu.CompilerParams(needs_layout_passes=False),
        out_type=jax.ShapeDtypeStruct((B, AD_, A_), jnp.float32),
        scratch_types=[
            pltpu.VMEM((CTPAD,), jnp.float32),
            pltpu.VMEM((AD, A), jnp.float32),
            pltpu.VMEM((AD, A), jnp.float32),
            pltpu.VMEM((AD, A), jnp.float32),
            pltpu.VMEM((AD, A), jnp.float32),
            pltpu.VMEM((AD, A), jnp.float32),
            pltpu.VMEM((AD, AP), jnp.float32),
            pltpu.VMEM((A,), jnp.int32),
            pltpu.VMEM((A,), jnp.int32),
            pltpu.VMEM((A, AP), jnp.int32),
            pltpu.VMEM((A, AP), jnp.int32),
            pltpu.VMEM((MAXC,), jnp.int32),
            pltpu.VMEM((MAXC,), jnp.int32),
        ],
    )(_sc_body)
    outt = run(ct, lgt, illt, q0t, py, px)
    return jnp.swapaxes(outt, 1, 2)
